# baseline (speedup 1.0000x reference)
import jax
import jax.numpy as jnp
from jax.experimental import pallas as pl
from jax.experimental.pallas import tpu as pltpu

VOCAB = 4096
MASK_RATE = 0.15
MASK_ID = 1
RANDOM_RATE = 0.1
CELLS = 512
CENTER_RATE = MASK_RATE / 27.0

BLOCK_B = 128


def _body(tok_ref, sel_ref, rep_ref, out_tok_ref, out_m_ref, out_w_ref):
    tok = tok_ref[...]
    sel = sel_ref[...]
    rep = rep_ref[...]

    lane = jax.lax.broadcasted_iota(jnp.int32, tok.shape, 1)
    z = lane & 7
    y = (lane >> 3) & 7
    x = lane >> 6

    selectable = tok != 0
    c = jnp.where(jnp.logical_and(sel < CENTER_RATE, selectable), 1, 0)

    zero = jnp.zeros_like(c)
    d = c | jnp.where(z > 0, pltpu.roll(c, 1, 1), zero)
    d = d | jnp.where(z < 7, pltpu.roll(c, CELLS - 1, 1), zero)
    dy = d | jnp.where(y > 0, pltpu.roll(d, 8, 1), zero)
    dy = dy | jnp.where(y < 7, pltpu.roll(d, CELLS - 8, 1), zero)
    dx = dy | jnp.where(x > 0, pltpu.roll(dy, 64, 1), zero)
    dx = dx | jnp.where(x < 7, pltpu.roll(dy, CELLS - 64, 1), zero)

    mask = jnp.logical_and(dx > 0, selectable)
    rand_ids = jnp.clip((rep * VOCAB).astype(jnp.int32), 0, VOCAB - 1)
    repl = jnp.where(rep < RANDOM_RATE, rand_ids,
                     jnp.full_like(tok, MASK_ID))
    out_tok_ref[...] = jnp.where(mask, repl, tok)
    out_m_ref[...] = mask
    out_w_ref[...] = mask.astype(jnp.float32)


def kernel(token_ids, selection_noise, replacement_noise):
    shape = token_ids.shape
    b = shape[0]
    tok = token_ids.reshape(b, CELLS)
    sel = selection_noise.reshape(b, CELLS)
    rep = replacement_noise.reshape(b, CELLS)

    grid = (b // BLOCK_B,)
    spec = pl.BlockSpec((BLOCK_B, CELLS), lambda i: (i, 0))
    out_tok, out_m, out_w = pl.pallas_call(
        _body,
        grid=grid,
        in_specs=[spec, spec, spec],
        out_specs=[spec, spec, spec],
        out_shape=[
            jax.ShapeDtypeStruct((b, CELLS), jnp.int32),
            jax.ShapeDtypeStruct((b, CELLS), jnp.bool_),
            jax.ShapeDtypeStruct((b, CELLS), jnp.float32),
        ],
    )(tok, sel, rep)

    return (out_tok.reshape(shape), out_m.reshape(shape),
            out_w.reshape(shape))


# speedup vs baseline: 1.0127x; 1.0127x over previous
import jax
import jax.numpy as jnp
from jax.experimental import pallas as pl
from jax.experimental.pallas import tpu as pltpu

VOCAB = 4096
MASK_RATE = 0.15
MASK_ID = 1
RANDOM_RATE = 0.1
CELLS = 512
CENTER_RATE = MASK_RATE / 27.0

BLOCK_B = 128


def _body(tok_ref, sel_ref, rep_ref, out_tok_ref, out_m_ref, out_w_ref):
    tok = tok_ref[...]
    sel = sel_ref[...]
    rep = rep_ref[...]

    lane = jax.lax.broadcasted_iota(jnp.int32, tok.shape, 1)
    z = lane & 7
    y = (lane >> 3) & 7
    x = lane >> 6

    selectable = tok != 0
    c = jnp.where(jnp.logical_and(sel < CENTER_RATE, selectable), 1, 0)

    zero = jnp.zeros_like(c)
    d = c | jnp.where(z > 0, pltpu.roll(c, 1, 1), zero)
    d = d | jnp.where(z < 7, pltpu.roll(c, CELLS - 1, 1), zero)
    dy = d | jnp.where(y > 0, pltpu.roll(d, 8, 1), zero)
    dy = dy | jnp.where(y < 7, pltpu.roll(d, CELLS - 8, 1), zero)
    dx = dy | jnp.where(x > 0, pltpu.roll(dy, 64, 1), zero)
    dx = dx | jnp.where(x < 7, pltpu.roll(dy, CELLS - 64, 1), zero)

    mask = jnp.logical_and(dx > 0, selectable)
    rand_ids = jnp.clip((rep * VOCAB).astype(jnp.int32), 0, VOCAB - 1)
    repl = jnp.where(rep < RANDOM_RATE, rand_ids,
                     jnp.full_like(tok, MASK_ID))
    out_tok_ref[...] = jnp.where(mask, repl, tok)
    out_m_ref[...] = mask.astype(jnp.int8)
    out_w_ref[...] = mask.astype(jnp.float32)


def kernel(token_ids, selection_noise, replacement_noise):
    shape = token_ids.shape
    b = shape[0]
    tok = token_ids.reshape(b, CELLS)
    sel = selection_noise.reshape(b, CELLS)
    rep = replacement_noise.reshape(b, CELLS)

    grid = (b // BLOCK_B,)
    spec = pl.BlockSpec((BLOCK_B, CELLS), lambda i: (i, 0))
    out_tok, out_m, out_w = pl.pallas_call(
        _body,
        grid=grid,
        in_specs=[spec, spec, spec],
        out_specs=[spec, spec, spec],
        out_shape=[
            jax.ShapeDtypeStruct((b, CELLS), jnp.int32),
            jax.ShapeDtypeStruct((b, CELLS), jnp.int8),
            jax.ShapeDtypeStruct((b, CELLS), jnp.float32),
        ],
    )(tok, sel, rep)

    return (out_tok.reshape(shape), out_m.reshape(shape).astype(jnp.bool_),
            out_w.reshape(shape))


# speedup vs baseline: 1.0274x; 1.0145x over previous
import jax
import jax.numpy as jnp
from jax.experimental import pallas as pl
from jax.experimental.pallas import tpu as pltpu

VOCAB = 4096
MASK_RATE = 0.15
MASK_ID = 1
RANDOM_RATE = 0.1
CELLS = 512
CENTER_RATE = MASK_RATE / 27.0

BLOCK_B = 128


def _body(tok_ref, sel_ref, rep_ref, out_tok_ref, out_m_ref):
    tok = tok_ref[...]
    sel = sel_ref[...]
    rep = rep_ref[...]

    lane = jax.lax.broadcasted_iota(jnp.int32, tok.shape, 1)
    z = lane & 7
    y = (lane >> 3) & 7
    x = lane >> 6

    selectable = tok != 0
    c = jnp.where(jnp.logical_and(sel < CENTER_RATE, selectable), 1, 0)

    zero = jnp.zeros_like(c)
    d = c | jnp.where(z > 0, pltpu.roll(c, 1, 1), zero)
    d = d | jnp.where(z < 7, pltpu.roll(c, CELLS - 1, 1), zero)
    dy = d | jnp.where(y > 0, pltpu.roll(d, 8, 1), zero)
    dy = dy | jnp.where(y < 7, pltpu.roll(d, CELLS - 8, 1), zero)
    dx = dy | jnp.where(x > 0, pltpu.roll(dy, 64, 1), zero)
    dx = dx | jnp.where(x < 7, pltpu.roll(dy, CELLS - 64, 1), zero)

    mask = jnp.logical_and(dx > 0, selectable)
    rand_ids = jnp.clip((rep * VOCAB).astype(jnp.int32), 0, VOCAB - 1)
    repl = jnp.where(rep < RANDOM_RATE, rand_ids,
                     jnp.full_like(tok, MASK_ID))
    out_tok_ref[...] = jnp.where(mask, repl, tok)
    out_m_ref[...] = mask.astype(jnp.int8)


def kernel(token_ids, selection_noise, replacement_noise):
    shape = token_ids.shape
    b = shape[0]
    tok = token_ids.reshape(b, CELLS)
    sel = selection_noise.reshape(b, CELLS)
    rep = replacement_noise.reshape(b, CELLS)

    grid = (b // BLOCK_B,)
    spec = pl.BlockSpec((BLOCK_B, CELLS), lambda i: (i, 0))
    out_tok, out_m = pl.pallas_call(
        _body,
        grid=grid,
        in_specs=[spec, spec, spec],
        out_specs=[spec, spec],
        out_shape=[
            jax.ShapeDtypeStruct((b, CELLS), jnp.int32),
            jax.ShapeDtypeStruct((b, CELLS), jnp.int8),
        ],
    )(tok, sel, rep)

    out_m = out_m.reshape(shape)
    return (out_tok.reshape(shape), out_m.astype(jnp.bool_),
            out_m.astype(jnp.float32))


# speedup vs baseline: 1.1841x; 1.1525x over previous
import jax
import jax.numpy as jnp
from jax.experimental import pallas as pl
from jax.experimental.pallas import tpu as pltpu

VOCAB = 4096
MASK_RATE = 0.15
MASK_ID = 1
RANDOM_RATE = 0.1
CELLS = 512
CENTER_RATE = MASK_RATE / 27.0

BLOCK_B = 256


def _body(tok_ref, sel_ref, rep_ref, out_tok_ref, out_w_ref):
    tok = tok_ref[...]
    sel = sel_ref[...]
    rep = rep_ref[...]

    lane = jax.lax.broadcasted_iota(jnp.int32, tok.shape, 1)
    z = lane & 7
    y = (lane >> 3) & 7
    x = lane >> 6

    selectable = tok != 0
    c = jnp.where(jnp.logical_and(sel < CENTER_RATE, selectable), 1, 0)

    zero = jnp.zeros_like(c)
    d = c | jnp.where(z > 0, pltpu.roll(c, 1, 1), zero)
    d = d | jnp.where(z < 7, pltpu.roll(c, CELLS - 1, 1), zero)
    dy = d | jnp.where(y > 0, pltpu.roll(d, 8, 1), zero)
    dy = dy | jnp.where(y < 7, pltpu.roll(d, CELLS - 8, 1), zero)
    dx = dy | jnp.where(x > 0, pltpu.roll(dy, 64, 1), zero)
    dx = dx | jnp.where(x < 7, pltpu.roll(dy, CELLS - 64, 1), zero)

    mask = jnp.logical_and(dx > 0, selectable)
    rand_ids = jnp.clip((rep * VOCAB).astype(jnp.int32), 0, VOCAB - 1)
    repl = jnp.where(rep < RANDOM_RATE, rand_ids,
                     jnp.full_like(tok, MASK_ID))
    out_tok_ref[...] = jnp.where(mask, repl, tok)
    out_w_ref[...] = mask.astype(jnp.float32)


def kernel(token_ids, selection_noise, replacement_noise):
    shape = token_ids.shape
    b = shape[0]
    tok = token_ids.reshape(b, CELLS)
    sel = selection_noise.reshape(b, CELLS)
    rep = replacement_noise.reshape(b, CELLS)

    grid = (b // BLOCK_B,)
    spec = pl.BlockSpec((BLOCK_B, CELLS), lambda i: (i, 0))
    out_tok, out_w = pl.pallas_call(
        _body,
        grid=grid,
        in_specs=[spec, spec, spec],
        out_specs=[spec, spec],
        out_shape=[
            jax.ShapeDtypeStruct((b, CELLS), jnp.int32),
            jax.ShapeDtypeStruct((b, CELLS), jnp.float32),
        ],
    )(tok, sel, rep)

    out_tok = out_tok.reshape(shape)
    out_w = out_w.reshape(shape)
    return out_tok, out_w.astype(jnp.bool_), out_w


# speedup vs baseline: 1.1912x; 1.0060x over previous
import jax
import jax.numpy as jnp
from jax.experimental import pallas as pl
from jax.experimental.pallas import tpu as pltpu

VOCAB = 4096
MASK_RATE = 0.15
MASK_ID = 1
RANDOM_RATE = 0.1
CELLS = 512
CENTER_RATE = MASK_RATE / 27.0

BLOCK_B = 512


def _body(tok_ref, sel_ref, rep_ref, out_tok_ref, out_w_ref):
    tok = tok_ref[...]
    sel = sel_ref[...]
    rep = rep_ref[...]

    lane = jax.lax.broadcasted_iota(jnp.int32, tok.shape, 1)
    z = lane & 7
    y = (lane >> 3) & 7
    x = lane >> 6

    selectable = tok != 0
    c = jnp.where(jnp.logical_and(sel < CENTER_RATE, selectable), 1, 0)

    zero = jnp.zeros_like(c)
    d = c | jnp.where(z > 0, pltpu.roll(c, 1, 1), zero)
    d = d | jnp.where(z < 7, pltpu.roll(c, CELLS - 1, 1), zero)
    dy = d | jnp.where(y > 0, pltpu.roll(d, 8, 1), zero)
    dy = dy | jnp.where(y < 7, pltpu.roll(d, CELLS - 8, 1), zero)
    dx = dy | jnp.where(x > 0, pltpu.roll(dy, 64, 1), zero)
    dx = dx | jnp.where(x < 7, pltpu.roll(dy, CELLS - 64, 1), zero)

    mask = jnp.logical_and(dx > 0, selectable)
    rand_ids = jnp.clip((rep * VOCAB).astype(jnp.int32), 0, VOCAB - 1)
    repl = jnp.where(rep < RANDOM_RATE, rand_ids,
                     jnp.full_like(tok, MASK_ID))
    out_tok_ref[...] = jnp.where(mask, repl, tok)
    out_w_ref[...] = mask.astype(jnp.float32)


def kernel(token_ids, selection_noise, replacement_noise):
    shape = token_ids.shape
    b = shape[0]
    tok = token_ids.reshape(b, CELLS)
    sel = selection_noise.reshape(b, CELLS)
    rep = replacement_noise.reshape(b, CELLS)

    grid = (b // BLOCK_B,)
    spec = pl.BlockSpec((BLOCK_B, CELLS), lambda i: (i, 0))
    out_tok, out_w = pl.pallas_call(
        _body,
        grid=grid,
        in_specs=[spec, spec, spec],
        out_specs=[spec, spec],
        out_shape=[
            jax.ShapeDtypeStruct((b, CELLS), jnp.int32),
            jax.ShapeDtypeStruct((b, CELLS), jnp.float32),
        ],
    )(tok, sel, rep)

    out_tok = out_tok.reshape(shape)
    out_w = out_w.reshape(shape)
    return out_tok, out_w.astype(jnp.bool_), out_w
